# Initial kernel scaffold; baseline (speedup 1.0000x reference)
import jax
import jax.numpy as jnp
from jax.experimental import pallas as pl


def kernel(symptoms_vector_tensor, batch_index, W1, b1, gamma, beta, running_mean, running_var, W2, b2):
    raise NotImplementedError("write your pallas kernel here")


# speedup vs baseline: 2.4909x; 2.4909x over previous
import functools

import jax
import jax.numpy as jnp
from jax import lax
from jax.experimental import pallas as pl
from jax.experimental.pallas import tpu as pltpu
from jax.experimental.pallas import tpu_sc as plsc

N_SEG = 10000
N_ELEM = 640000
UMAP_DIM = 100
HIDDEN = 256
NUM_DX = 1000

NC = 2
NS = 16
NW = NC * NS
ROWS_PER_W = N_ELEM // NW
SUB = 100
NSUB = 2
BLOCK = SUB * NSUB
IGRP = 8
GROUP = SUB * IGRP
NITER = ROWS_PER_W // GROUP
RD_WORKERS = 10
RD_ROWS = N_SEG // RD_WORKERS
DPAD = 128


def _seg_sum_body(x_hbm, bi_hbm, z_hbm, out_hbm, xv, iv, shared):
    cid = lax.axis_index("c")
    sid = lax.axis_index("s")
    wid = cid * NS + sid

    @pl.when(sid < RD_WORKERS)
    def _():
        pltpu.sync_copy(z_hbm, shared.at[pl.ds(sid * RD_ROWS, RD_ROWS)])

    plsc.subcore_barrier()

    row0 = wid * ROWS_PER_W
    irow0 = wid * (ROWS_PER_W // SUB)

    def body(i, _):
        pltpu.sync_copy(bi_hbm.at[pl.ds(irow0 + i * IGRP, IGRP)], iv)
        for h in range(IGRP // NSUB):
            pltpu.sync_copy(
                x_hbm.at[pl.ds(row0 + i * GROUP + h * BLOCK, BLOCK)], xv)
            for j in range(NSUB):
                pltpu.sync_copy(xv.at[pl.ds(j * SUB, SUB)],
                                shared.at[iv.at[h * NSUB + j]], add=True)
        return 0

    lax.fori_loop(0, NITER, body, 0)

    plsc.subcore_barrier()

    @pl.when(sid < RD_WORKERS)
    def _():
        pltpu.sync_copy(shared.at[pl.ds(sid * RD_ROWS, RD_ROWS)],
                        out_hbm.at[cid, pl.ds(sid * RD_ROWS, RD_ROWS)])


@jax.jit
def _seg_sum_sc(x, bi2, zeros):
    mesh = plsc.VectorSubcoreMesh(core_axis_name="c", subcore_axis_name="s")
    return pl.kernel(
        _seg_sum_body,
        out_type=jax.ShapeDtypeStruct((NC, N_SEG, DPAD), jnp.float32),
        mesh=mesh,
        scratch_types=[
            pltpu.VMEM((BLOCK, DPAD), jnp.float32),
            pltpu.VMEM((IGRP, SUB), jnp.int32),
            pltpu.VMEM_SHARED((N_SEG, DPAD), jnp.float32),
        ],
    )(x, bi2, zeros)


def _mlp_body(p_ref, w1_ref, b1_ref, g_ref, bt_ref, mu_ref, var_ref,
              w2_ref, b2_ref, out_ref):
    pooled = p_ref[0] + p_ref[1]
    h = lax.dot_general(pooled, w1_ref[...], (((1,), (1,)), ((), ())),
                        preferred_element_type=jnp.float32)
    h = h + b1_ref[...]
    inv = lax.rsqrt(var_ref[...] + 1e-5)
    h = (h - mu_ref[...]) * (inv * g_ref[...]) + bt_ref[...]
    h = jnp.where(h >= 0, h, 0.01 * h)
    out_ref[...] = lax.dot_general(h, w2_ref[...], (((1,), (1,)), ((), ())),
                                   preferred_element_type=jnp.float32) + b2_ref[...]


@jax.jit
def _mlp_tc(p2, W1, b1, gamma, beta, mu, var, W2, b2):
    mb = 1000
    grid = N_SEG // mb
    return pl.pallas_call(
        _mlp_body,
        grid=(grid,),
        in_specs=[
            pl.BlockSpec((NC, mb, DPAD), lambda i: (0, i, 0)),
            pl.BlockSpec((HIDDEN, DPAD), lambda i: (0, 0)),
            pl.BlockSpec((1, HIDDEN), lambda i: (0, 0)),
            pl.BlockSpec((1, HIDDEN), lambda i: (0, 0)),
            pl.BlockSpec((1, HIDDEN), lambda i: (0, 0)),
            pl.BlockSpec((1, HIDDEN), lambda i: (0, 0)),
            pl.BlockSpec((1, HIDDEN), lambda i: (0, 0)),
            pl.BlockSpec((NUM_DX, HIDDEN), lambda i: (0, 0)),
            pl.BlockSpec((1, NUM_DX), lambda i: (0, 0)),
        ],
        out_specs=pl.BlockSpec((mb, NUM_DX), lambda i: (i, 0)),
        out_shape=jax.ShapeDtypeStruct((N_SEG, NUM_DX), jnp.float32),
        compiler_params=pltpu.CompilerParams(
            dimension_semantics=("parallel",)),
    )(p2, W1, b1, gamma, beta, mu, var, W2, b2)


def kernel(symptoms_vector_tensor, batch_index, W1, b1, gamma, beta,
           running_mean, running_var, W2, b2):
    x128 = jnp.pad(symptoms_vector_tensor, ((0, 0), (0, DPAD - UMAP_DIM)))
    W1p = jnp.pad(W1, ((0, 0), (0, DPAD - UMAP_DIM)))
    bi2 = batch_index.astype(jnp.int32).reshape(N_ELEM // SUB, SUB)
    zeros = jnp.zeros((RD_ROWS, DPAD), jnp.float32)
    partials = _seg_sum_sc(x128, bi2, zeros)
    return _mlp_tc(partials, W1p, b1.reshape(1, -1), gamma.reshape(1, -1),
                   beta.reshape(1, -1), running_mean.reshape(1, -1),
                   running_var.reshape(1, -1), W2, b2.reshape(1, -1))


# speedup vs baseline: 2.7948x; 1.1220x over previous
import functools

import jax
import jax.numpy as jnp
from jax import lax
from jax.experimental import pallas as pl
from jax.experimental.pallas import tpu as pltpu
from jax.experimental.pallas import tpu_sc as plsc

N_SEG = 10000
N_ELEM = 640000
UMAP_DIM = 100
HIDDEN = 256
NUM_DX = 1000

NC = 2
NS = 16
NW = NC * NS
ROWS_PER_W = N_ELEM // NW
SUB = 80
BLOCK = 2 * SUB
NBLK_REAL = ROWS_PER_W // BLOCK
IROWS = ROWS_PER_W // SUB
IROWS_PAD = 256
IGRP = 8
NFETCH = IROWS_PAD // IGRP
NBLK = NFETCH * (IGRP // 2)
RD_WORKERS = 10
RD_ROWS = N_SEG // RD_WORKERS
DPAD = 128
DUMP_SEG = N_SEG
SHARED_ROWS = N_SEG + 16


def _seg_sum_body(x_hbm, bi_hbm, z_hbm, out_hbm, xv0, xv1, iv, shared,
                  sem0, sem1):
    cid = lax.axis_index("c")
    sid = lax.axis_index("s")
    wid = cid * NS + sid
    bufs = (xv0, xv1)
    sems = (sem0, sem1)

    @pl.when(sid < RD_WORKERS)
    def _():
        pltpu.sync_copy(z_hbm, shared.at[pl.ds(sid * RD_ROWS, RD_ROWS)])

    plsc.subcore_barrier()

    row0 = wid * ROWS_PER_W

    pltpu.async_copy(x_hbm.at[pl.ds(row0, BLOCK)], bufs[0], sems[0])

    def outer(f, _):
        pltpu.sync_copy(bi_hbm.at[wid, pl.ds(f * IGRP, IGRP)], iv)
        for g in range(IGRP // 2):
            p = g % 2
            b = f * (IGRP // 2) + g
            nxt = jnp.minimum(b + 1, NBLK_REAL - 1)

            @pl.when(b + 1 < NBLK)
            def _():
                pltpu.async_copy(x_hbm.at[pl.ds(row0 + nxt * BLOCK, BLOCK)],
                                 bufs[1 - p], sems[1 - p])

            pltpu.make_async_copy(
                x_hbm.at[pl.ds(row0, BLOCK)], bufs[p], sems[p]).wait()
            for j in range(2):
                pltpu.sync_copy(bufs[p].at[pl.ds(j * SUB, SUB)],
                                shared.at[iv.at[2 * g + j]], add=True)
        return 0

    lax.fori_loop(0, NFETCH, outer, 0)

    plsc.subcore_barrier()

    @pl.when(sid < RD_WORKERS)
    def _():
        pltpu.sync_copy(shared.at[pl.ds(sid * RD_ROWS, RD_ROWS)],
                        out_hbm.at[cid, pl.ds(sid * RD_ROWS, RD_ROWS)])


@jax.jit
def _seg_sum_sc(x, bi3, zeros):
    mesh = plsc.VectorSubcoreMesh(core_axis_name="c", subcore_axis_name="s")
    return pl.kernel(
        _seg_sum_body,
        out_type=jax.ShapeDtypeStruct((NC, N_SEG, DPAD), jnp.float32),
        mesh=mesh,
        scratch_types=[
            pltpu.VMEM((BLOCK, DPAD), jnp.float32),
            pltpu.VMEM((BLOCK, DPAD), jnp.float32),
            pltpu.VMEM((IGRP, SUB), jnp.int32),
            pltpu.VMEM_SHARED((SHARED_ROWS, DPAD), jnp.float32),
            pltpu.SemaphoreType.DMA,
            pltpu.SemaphoreType.DMA,
        ],
    )(x, bi3, zeros)


def _mlp_body(p_ref, w1_ref, b1_ref, g_ref, bt_ref, mu_ref, var_ref,
              w2_ref, b2_ref, out_ref):
    pooled = p_ref[0] + p_ref[1]
    h = lax.dot_general(pooled, w1_ref[...], (((1,), (1,)), ((), ())),
                        preferred_element_type=jnp.float32)
    h = h + b1_ref[...]
    inv = lax.rsqrt(var_ref[...] + 1e-5)
    h = (h - mu_ref[...]) * (inv * g_ref[...]) + bt_ref[...]
    h = jnp.where(h >= 0, h, 0.01 * h)
    out_ref[...] = lax.dot_general(h, w2_ref[...], (((1,), (1,)), ((), ())),
                                   preferred_element_type=jnp.float32) + b2_ref[...]


@jax.jit
def _mlp_tc(p2, W1, b1, gamma, beta, mu, var, W2, b2):
    mb = 1000
    grid = N_SEG // mb
    return pl.pallas_call(
        _mlp_body,
        grid=(grid,),
        in_specs=[
            pl.BlockSpec((NC, mb, DPAD), lambda i: (0, i, 0)),
            pl.BlockSpec((HIDDEN, DPAD), lambda i: (0, 0)),
            pl.BlockSpec((1, HIDDEN), lambda i: (0, 0)),
            pl.BlockSpec((1, HIDDEN), lambda i: (0, 0)),
            pl.BlockSpec((1, HIDDEN), lambda i: (0, 0)),
            pl.BlockSpec((1, HIDDEN), lambda i: (0, 0)),
            pl.BlockSpec((1, HIDDEN), lambda i: (0, 0)),
            pl.BlockSpec((NUM_DX, HIDDEN), lambda i: (0, 0)),
            pl.BlockSpec((1, NUM_DX), lambda i: (0, 0)),
        ],
        out_specs=pl.BlockSpec((mb, NUM_DX), lambda i: (i, 0)),
        out_shape=jax.ShapeDtypeStruct((N_SEG, NUM_DX), jnp.float32),
        compiler_params=pltpu.CompilerParams(
            dimension_semantics=("parallel",)),
    )(p2, W1, b1, gamma, beta, mu, var, W2, b2)


def kernel(symptoms_vector_tensor, batch_index, W1, b1, gamma, beta,
           running_mean, running_var, W2, b2):
    x128 = jnp.pad(symptoms_vector_tensor, ((0, 0), (0, DPAD - UMAP_DIM)))
    W1p = jnp.pad(W1, ((0, 0), (0, DPAD - UMAP_DIM)))
    bi2 = batch_index.astype(jnp.int32).reshape(NW, IROWS, SUB)
    bi3 = jnp.concatenate(
        [bi2, jnp.full((NW, IROWS_PAD - IROWS, SUB), DUMP_SEG, jnp.int32)],
        axis=1)
    zeros = jnp.zeros((RD_ROWS, DPAD), jnp.float32)
    partials = _seg_sum_sc(x128, bi3, zeros)
    return _mlp_tc(partials, W1p, b1.reshape(1, -1), gamma.reshape(1, -1),
                   beta.reshape(1, -1), running_mean.reshape(1, -1),
                   running_var.reshape(1, -1), W2, b2.reshape(1, -1))


# speedup vs baseline: 2.8198x; 1.0090x over previous
import functools

import jax
import jax.numpy as jnp
from jax import lax
from jax.experimental import pallas as pl
from jax.experimental.pallas import tpu as pltpu
from jax.experimental.pallas import tpu_sc as plsc

N_SEG = 10000
N_ELEM = 640000
UMAP_DIM = 100
HIDDEN = 256
NUM_DX = 1000

NC = 2
NS = 16
NW = NC * NS
SUB = 80
BLOCK = 2 * SUB
IGRP = 8
ROWS_BIG = 20480
NFETCH_BIG = ROWS_BIG // (SUB * IGRP)
NFETCH_LAST = (N_ELEM - 31 * ROWS_BIG) // (SUB * IGRP)
RD_WORKERS = 10
RD_ROWS = N_SEG // RD_WORKERS
DPAD = 128


def _seg_sum_body(x_hbm, bi_hbm, z_hbm, out_hbm, xv0, xv1, iv, shared,
                  sem0, sem1, ssem0, ssem1):
    cid = lax.axis_index("c")
    sid = lax.axis_index("s")
    wid = cid * NS + sid
    bufs = (xv0, xv1)
    sems = (sem0, sem1)
    ssems = (ssem0, ssem1)

    @pl.when(sid < RD_WORKERS)
    def _():
        pltpu.sync_copy(z_hbm, shared.at[pl.ds(sid * RD_ROWS, RD_ROWS)])

    plsc.subcore_barrier()

    row0 = wid * ROWS_BIG
    irow0 = wid * (ROWS_BIG // SUB)
    nfetch = jnp.where(wid == NW - 1, NFETCH_LAST, NFETCH_BIG)
    nblk = nfetch * (IGRP // 2)

    pltpu.async_copy(x_hbm.at[pl.ds(row0, BLOCK)], bufs[0], sems[0])

    def scatter_pair(p, g):
        for j in range(2):
            pltpu.async_copy(bufs[p].at[pl.ds(j * SUB, SUB)],
                             shared.at[iv.at[2 * g + j]], ssems[p],
                             add=True)

    def drain_pair(p, g):
        for j in range(2):
            pltpu.make_async_copy(bufs[p].at[pl.ds(j * SUB, SUB)],
                                  shared.at[iv.at[2 * g + j]],
                                  ssems[p]).wait()

    def outer(f, _):
        @pl.when(f > 0)
        def _():
            drain_pair(1, IGRP // 2 - 1)

        pltpu.sync_copy(bi_hbm.at[pl.ds(irow0 + f * IGRP, IGRP)], iv)
        for g in range(IGRP // 2):
            p = g % 2
            b = f * (IGRP // 2) + g

            if g > 0:
                drain_pair(1 - p, g - 1)

            @pl.when(b + 1 < nblk)
            def _():
                pltpu.async_copy(x_hbm.at[pl.ds(row0 + (b + 1) * BLOCK,
                                                BLOCK)],
                                 bufs[1 - p], sems[1 - p])

            pltpu.make_async_copy(
                x_hbm.at[pl.ds(row0, BLOCK)], bufs[p], sems[p]).wait()
            scatter_pair(p, g)
        return 0

    lax.fori_loop(0, nfetch, outer, 0)

    drain_pair(1, IGRP // 2 - 1)

    plsc.subcore_barrier()

    @pl.when(sid < RD_WORKERS)
    def _():
        pltpu.sync_copy(shared.at[pl.ds(sid * RD_ROWS, RD_ROWS)],
                        out_hbm.at[cid, pl.ds(sid * RD_ROWS, RD_ROWS)])


@jax.jit
def _seg_sum_sc(x, bi2, zeros):
    mesh = plsc.VectorSubcoreMesh(core_axis_name="c", subcore_axis_name="s")
    return pl.kernel(
        _seg_sum_body,
        out_type=jax.ShapeDtypeStruct((NC, N_SEG, DPAD), jnp.float32),
        mesh=mesh,
        scratch_types=[
            pltpu.VMEM((BLOCK, DPAD), jnp.float32),
            pltpu.VMEM((BLOCK, DPAD), jnp.float32),
            pltpu.VMEM((IGRP, SUB), jnp.int32),
            pltpu.VMEM_SHARED((N_SEG, DPAD), jnp.float32),
            pltpu.SemaphoreType.DMA,
            pltpu.SemaphoreType.DMA,
            pltpu.SemaphoreType.DMA,
            pltpu.SemaphoreType.DMA,
        ],
    )(x, bi2, zeros)


def _mlp_body(p_ref, w1_ref, b1_ref, g_ref, bt_ref, mu_ref, var_ref,
              w2_ref, b2_ref, out_ref):
    pooled = p_ref[0] + p_ref[1]
    h = lax.dot_general(pooled, w1_ref[...], (((1,), (1,)), ((), ())),
                        preferred_element_type=jnp.float32)
    h = h + b1_ref[...]
    inv = lax.rsqrt(var_ref[...] + 1e-5)
    h = (h - mu_ref[...]) * (inv * g_ref[...]) + bt_ref[...]
    h = jnp.where(h >= 0, h, 0.01 * h)
    out_ref[...] = lax.dot_general(h, w2_ref[...], (((1,), (1,)), ((), ())),
                                   preferred_element_type=jnp.float32) + b2_ref[...]


@jax.jit
def _mlp_tc(p2, W1, b1, gamma, beta, mu, var, W2, b2):
    mb = 2000
    grid = N_SEG // mb
    return pl.pallas_call(
        _mlp_body,
        grid=(grid,),
        in_specs=[
            pl.BlockSpec((NC, mb, DPAD), lambda i: (0, i, 0)),
            pl.BlockSpec((HIDDEN, DPAD), lambda i: (0, 0)),
            pl.BlockSpec((1, HIDDEN), lambda i: (0, 0)),
            pl.BlockSpec((1, HIDDEN), lambda i: (0, 0)),
            pl.BlockSpec((1, HIDDEN), lambda i: (0, 0)),
            pl.BlockSpec((1, HIDDEN), lambda i: (0, 0)),
            pl.BlockSpec((1, HIDDEN), lambda i: (0, 0)),
            pl.BlockSpec((NUM_DX, HIDDEN), lambda i: (0, 0)),
            pl.BlockSpec((1, NUM_DX), lambda i: (0, 0)),
        ],
        out_specs=pl.BlockSpec((mb, NUM_DX), lambda i: (i, 0)),
        out_shape=jax.ShapeDtypeStruct((N_SEG, NUM_DX), jnp.float32),
        compiler_params=pltpu.CompilerParams(
            dimension_semantics=("parallel",)),
    )(p2, W1, b1, gamma, beta, mu, var, W2, b2)


def kernel(symptoms_vector_tensor, batch_index, W1, b1, gamma, beta,
           running_mean, running_var, W2, b2):
    x128 = jnp.pad(symptoms_vector_tensor, ((0, 0), (0, DPAD - UMAP_DIM)))
    W1p = jnp.pad(W1, ((0, 0), (0, DPAD - UMAP_DIM)))
    bi2 = batch_index.astype(jnp.int32).reshape(N_ELEM // SUB, SUB)
    zeros = jnp.zeros((RD_ROWS, DPAD), jnp.float32)
    partials = _seg_sum_sc(x128, bi2, zeros)
    return _mlp_tc(partials, W1p, b1.reshape(1, -1), gamma.reshape(1, -1),
                   beta.reshape(1, -1), running_mean.reshape(1, -1),
                   running_var.reshape(1, -1), W2, b2.reshape(1, -1))
